# baseline (speedup 1.0000x reference)
import jax
import jax.numpy as jnp
from jax.experimental import pallas as pl

_LST = 64
_ITERS_INST = 8
_ITERS_TIME = 9


def _fused_kernel(x_ref, o_ref):
    x = x_ref[0]
    T = x.shape[1]
    nseg = T // _LST
    dt = x.dtype

    rM = jax.lax.broadcasted_iota(jnp.int32, (T, nseg), 0) // _LST
    cM = jax.lax.broadcasted_iota(jnp.int32, (T, nseg), 1)
    M = (rM == cM).astype(dt)
    rT = jax.lax.broadcasted_iota(jnp.int32, (nseg, T), 0)
    cT = jax.lax.broadcasted_iota(jnp.int32, (nseg, T), 1) // _LST
    Mt = (rT == cT).astype(dt)

    def dot(a, b):
        return jax.lax.dot(a, b, preferred_element_type=jnp.float32,
                           precision=jax.lax.Precision.HIGHEST)

    tau_i = jnp.max(x, axis=0, keepdims=True) - 1.0

    def body_i(_, tau):
        mask = (x > tau).astype(dt)
        S = jnp.sum(x * mask, axis=0, keepdims=True)
        C = jnp.sum(mask, axis=0, keepdims=True)
        return jnp.where(C > 0.0, (S - 1.0) / jnp.maximum(C, 1.0), tau)

    tau_i = jax.lax.fori_loop(0, _ITERS_INST, body_i, tau_i)

    tau_t = (dot(x, M) - 1.0) / jnp.float32(_LST)

    def body_t(_, tau):
        tau_b = dot(tau, Mt)
        mask = (x > tau_b).astype(dt)
        S = dot(x * mask, M)
        C = dot(mask, M)
        return jnp.where(C > 0.0, (S - 1.0) / jnp.maximum(C, 1.0), tau)

    tau_t = jax.lax.fori_loop(0, _ITERS_TIME, body_t, tau_t)
    tau_tb = dot(tau_t, Mt)

    o_ref[0] = jnp.maximum(x - tau_i, 0.0) * jnp.maximum(x - tau_tb, 0.0)


def kernel(midis_out):
    batch, two, n_insts, time = midis_out.shape
    assert time % _LST == 0

    bc = batch * two
    x3 = midis_out.reshape(bc, n_insts, time)

    T_BLK = 512
    out = pl.pallas_call(
        _fused_kernel,
        grid=(bc, time // T_BLK),
        in_specs=[pl.BlockSpec((1, n_insts, T_BLK), lambda i, j: (i, 0, j))],
        out_specs=pl.BlockSpec((1, n_insts, T_BLK), lambda i, j: (i, 0, j)),
        out_shape=jax.ShapeDtypeStruct(x3.shape, x3.dtype),
    )(x3)

    return out.reshape(batch, two, n_insts, time)


# speedup vs baseline: 2.3890x; 2.3890x over previous
import jax
import jax.numpy as jnp
from jax.experimental import pallas as pl

_LST = 64
_ITERS_INST = 6
_ITERS_TIME = 7


def _bf16_split(v):
    hi = v.astype(jnp.bfloat16).astype(jnp.float32)
    return hi, v - hi


def _fused_kernel(x_ref, o_ref):
    x = x_ref[0]
    T = x.shape[1]
    nseg = T // _LST
    dt = x.dtype

    rM = jax.lax.broadcasted_iota(jnp.int32, (T, nseg), 0) // _LST
    cM = jax.lax.broadcasted_iota(jnp.int32, (T, nseg), 1)
    M = (rM == cM).astype(dt)
    rT = jax.lax.broadcasted_iota(jnp.int32, (nseg, T), 0)
    cT = jax.lax.broadcasted_iota(jnp.int32, (nseg, T), 1) // _LST
    Mt = (rT == cT).astype(dt)

    def dot(a, b):
        return jax.lax.dot(a, b, preferred_element_type=jnp.float32)

    def dot_split(a, b):
        hi, lo = _bf16_split(a)
        return dot(hi, b) + dot(lo, b)

    tau_i = jnp.max(x, axis=0, keepdims=True) - 1.0

    def body_i(_, tau):
        mask = (x > tau).astype(dt)
        S = jnp.sum(x * mask, axis=0, keepdims=True)
        C = jnp.sum(mask, axis=0, keepdims=True)
        return jnp.where(C > 0.0, (S - 1.0) / jnp.maximum(C, 1.0), tau)

    tau_i = jax.lax.fori_loop(0, _ITERS_INST, body_i, tau_i)

    xb, xr = _bf16_split(x)
    tau_t = (dot(xb, M) + dot(xr, M) - 1.0) / jnp.float32(_LST)

    def body_t(_, tau):
        tau_b = dot_split(tau, Mt)
        mask = (x > tau_b).astype(dt)
        S = dot(xb * mask, M) + dot(xr * mask, M)
        C = dot(mask, M)
        return jnp.where(C > 0.0, (S - 1.0) / jnp.maximum(C, 1.0), tau)

    tau_t = jax.lax.fori_loop(0, _ITERS_TIME, body_t, tau_t)
    tau_tb = dot_split(tau_t, Mt)

    o_ref[0] = jnp.maximum(x - tau_i, 0.0) * jnp.maximum(x - tau_tb, 0.0)


def kernel(midis_out):
    batch, two, n_insts, time = midis_out.shape
    assert time % _LST == 0

    bc = batch * two
    x3 = midis_out.reshape(bc, n_insts, time)

    T_BLK = 512
    out = pl.pallas_call(
        _fused_kernel,
        grid=(bc, time // T_BLK),
        in_specs=[pl.BlockSpec((1, n_insts, T_BLK), lambda i, j: (i, 0, j))],
        out_specs=pl.BlockSpec((1, n_insts, T_BLK), lambda i, j: (i, 0, j)),
        out_shape=jax.ShapeDtypeStruct(x3.shape, x3.dtype),
    )(x3)

    return out.reshape(batch, two, n_insts, time)


# speedup vs baseline: 2.7649x; 1.1573x over previous
import jax
import jax.numpy as jnp
from jax.experimental import pallas as pl

_LST = 64
_ITERS_INST = 6
_ITERS_TIME = 7


def _bf16_split(v):
    hi = v.astype(jnp.bfloat16).astype(jnp.float32)
    return hi, v - hi


def _fused_kernel(x_ref, o_ref):
    x = x_ref[0]
    T = x.shape[1]
    nseg = T // _LST
    dt = x.dtype

    rM = jax.lax.broadcasted_iota(jnp.int32, (T, nseg), 0) // _LST
    cM = jax.lax.broadcasted_iota(jnp.int32, (T, nseg), 1)
    M = (rM == cM).astype(dt)
    rT = jax.lax.broadcasted_iota(jnp.int32, (nseg, T), 0)
    cT = jax.lax.broadcasted_iota(jnp.int32, (nseg, T), 1) // _LST
    Mt = (rT == cT).astype(dt)

    def dot(a, b):
        return jax.lax.dot(a, b, preferred_element_type=jnp.float32)

    def dot_split(a, b):
        hi, lo = _bf16_split(a)
        return dot(hi, b) + dot(lo, b)

    xb, xr = _bf16_split(x)
    tau_i = jnp.max(x, axis=0, keepdims=True) - 1.0
    tau_t = (dot(xb, M) + dot(xr, M) - 1.0) / jnp.float32(_LST)

    for it in range(max(_ITERS_INST, _ITERS_TIME)):
        if it < _ITERS_INST:
            mask = (x > tau_i).astype(dt)
            S = jnp.sum(x * mask, axis=0, keepdims=True)
            C = jnp.sum(mask, axis=0, keepdims=True)
            tau_i = jnp.where(C > 0.0, (S - 1.0) / jnp.maximum(C, 1.0), tau_i)
        if it < _ITERS_TIME:
            tau_b = dot_split(tau_t, Mt)
            mask = (x > tau_b).astype(dt)
            S = dot(xb * mask, M) + dot(xr * mask, M)
            C = dot(mask, M)
            tau_t = jnp.where(C > 0.0, (S - 1.0) / jnp.maximum(C, 1.0), tau_t)

    tau_tb = dot_split(tau_t, Mt)

    o_ref[0] = jnp.maximum(x - tau_i, 0.0) * jnp.maximum(x - tau_tb, 0.0)


def kernel(midis_out):
    batch, two, n_insts, time = midis_out.shape
    assert time % _LST == 0

    bc = batch * two
    x3 = midis_out.reshape(bc, n_insts, time)

    T_BLK = 512
    out = pl.pallas_call(
        _fused_kernel,
        grid=(bc, time // T_BLK),
        in_specs=[pl.BlockSpec((1, n_insts, T_BLK), lambda i, j: (i, 0, j))],
        out_specs=pl.BlockSpec((1, n_insts, T_BLK), lambda i, j: (i, 0, j)),
        out_shape=jax.ShapeDtypeStruct(x3.shape, x3.dtype),
    )(x3)

    return out.reshape(batch, two, n_insts, time)


# speedup vs baseline: 3.7314x; 1.3496x over previous
import jax
import jax.numpy as jnp
from jax.experimental import pallas as pl

_LST = 64
_ITERS_INST = 6
_ITERS_TIME = 7


def _bf16_split(v):
    hi = v.astype(jnp.bfloat16).astype(jnp.float32)
    return hi, v - hi


def _fused_kernel(x_ref, o_ref):
    x = x_ref[0]
    T = x.shape[1]
    nseg = T // _LST
    dt = x.dtype

    rM = jax.lax.broadcasted_iota(jnp.int32, (T, nseg), 0) // _LST
    cM = jax.lax.broadcasted_iota(jnp.int32, (T, nseg), 1)
    M = (rM == cM).astype(dt)
    rT = jax.lax.broadcasted_iota(jnp.int32, (nseg, T), 0)
    cT = jax.lax.broadcasted_iota(jnp.int32, (nseg, T), 1) // _LST
    Mt = (rT == cT).astype(dt)

    def dot(a, b):
        return jax.lax.dot(a, b, preferred_element_type=jnp.float32)

    def dot_split(a, b):
        hi, lo = _bf16_split(a)
        return dot(hi, b) + dot(lo, b)

    xb, xr = _bf16_split(x)
    tau_i = jnp.max(x, axis=0, keepdims=True) - 1.0
    tau_t = (dot(xb, M) + dot(xr, M) - 1.0) / jnp.float32(_LST)

    for it in range(max(_ITERS_INST, _ITERS_TIME)):
        if it < _ITERS_INST:
            mask = (x > tau_i).astype(dt)
            S = jnp.sum(x * mask, axis=0, keepdims=True)
            C = jnp.sum(mask, axis=0, keepdims=True)
            tau_i = jnp.where(C > 0.0, (S - 1.0) / jnp.maximum(C, 1.0), tau_i)
        if it < _ITERS_TIME:
            tau_b = dot_split(tau_t, Mt)
            mask = (x > tau_b).astype(dt)
            S = dot(xb * mask, M) + dot(xr * mask, M)
            C = dot(mask, M)
            tau_t = jnp.where(C > 0.0, (S - 1.0) / jnp.maximum(C, 1.0), tau_t)

    tau_tb = dot_split(tau_t, Mt)

    o_ref[0] = jnp.maximum(x - tau_i, 0.0) * jnp.maximum(x - tau_tb, 0.0)


def kernel(midis_out):
    batch, two, n_insts, time = midis_out.shape
    assert time % _LST == 0

    bc = batch * two
    x3 = midis_out.reshape(bc, n_insts, time)

    T_BLK = 1024
    out = pl.pallas_call(
        _fused_kernel,
        grid=(bc, time // T_BLK),
        in_specs=[pl.BlockSpec((1, n_insts, T_BLK), lambda i, j: (i, 0, j))],
        out_specs=pl.BlockSpec((1, n_insts, T_BLK), lambda i, j: (i, 0, j)),
        out_shape=jax.ShapeDtypeStruct(x3.shape, x3.dtype),
    )(x3)

    return out.reshape(batch, two, n_insts, time)


# speedup vs baseline: 4.5571x; 1.2213x over previous
import jax
import jax.numpy as jnp
from jax.experimental import pallas as pl

_LST = 64
_ITERS_INST = 6
_ITERS_TIME = 7


def _bf16_split(v):
    hi = v.astype(jnp.bfloat16).astype(jnp.float32)
    return hi, v - hi


def _fused_kernel(x_ref, o_ref):
    x = x_ref[0]
    T = x.shape[1]
    nseg = T // _LST
    dt = x.dtype

    rM = jax.lax.broadcasted_iota(jnp.int32, (T, nseg), 0) // _LST
    cM = jax.lax.broadcasted_iota(jnp.int32, (T, nseg), 1)
    M = (rM == cM).astype(dt)
    rT = jax.lax.broadcasted_iota(jnp.int32, (nseg, T), 0)
    cT = jax.lax.broadcasted_iota(jnp.int32, (nseg, T), 1) // _LST
    Mt = (rT == cT).astype(dt)

    def dot(a, b):
        return jax.lax.dot(a, b, preferred_element_type=jnp.float32)

    def dot_split(a, b):
        hi, lo = _bf16_split(a)
        return dot(hi, b) + dot(lo, b)

    xb, xr = _bf16_split(x)
    tau_i = jnp.max(x, axis=0, keepdims=True) - 1.0
    tau_t = (dot(xb, M) + dot(xr, M) - 1.0) / jnp.float32(_LST)

    for it in range(max(_ITERS_INST, _ITERS_TIME)):
        if it < _ITERS_INST:
            mask = (x > tau_i).astype(dt)
            S = jnp.sum(x * mask, axis=0, keepdims=True)
            C = jnp.sum(mask, axis=0, keepdims=True)
            tau_i = jnp.where(C > 0.0, (S - 1.0) / jnp.maximum(C, 1.0), tau_i)
        if it < _ITERS_TIME:
            tau_b = dot_split(tau_t, Mt)
            mask = (x > tau_b).astype(dt)
            S = dot(xb * mask, M) + dot(xr * mask, M)
            C = dot(mask, M)
            tau_t = jnp.where(C > 0.0, (S - 1.0) / jnp.maximum(C, 1.0), tau_t)

    tau_tb = dot_split(tau_t, Mt)

    o_ref[0] = jnp.maximum(x - tau_i, 0.0) * jnp.maximum(x - tau_tb, 0.0)


def kernel(midis_out):
    batch, two, n_insts, time = midis_out.shape
    assert time % _LST == 0

    bc = batch * two
    x3 = midis_out.reshape(bc, n_insts, time)

    T_BLK = 2048
    out = pl.pallas_call(
        _fused_kernel,
        grid=(bc, time // T_BLK),
        in_specs=[pl.BlockSpec((1, n_insts, T_BLK), lambda i, j: (i, 0, j))],
        out_specs=pl.BlockSpec((1, n_insts, T_BLK), lambda i, j: (i, 0, j)),
        out_shape=jax.ShapeDtypeStruct(x3.shape, x3.dtype),
    )(x3)

    return out.reshape(batch, two, n_insts, time)


# speedup vs baseline: 4.7551x; 1.0434x over previous
import jax
import jax.numpy as jnp
from jax.experimental import pallas as pl

_LST = 64
_ITERS_INST = 6
_ITERS_TIME = 7


def _bf16_split(v):
    hi = v.astype(jnp.bfloat16).astype(jnp.float32)
    return hi, v - hi


def _fused_kernel(x_ref, o_ref):
    x = x_ref[0]
    T = x.shape[1]
    nseg = T // _LST
    dt = x.dtype

    rM = jax.lax.broadcasted_iota(jnp.int32, (T, nseg), 0) // _LST
    cM = jax.lax.broadcasted_iota(jnp.int32, (T, nseg), 1)
    M = (rM == cM).astype(dt)
    rT = jax.lax.broadcasted_iota(jnp.int32, (nseg, T), 0)
    cT = jax.lax.broadcasted_iota(jnp.int32, (nseg, T), 1) // _LST
    Mt = (rT == cT).astype(dt)

    def dot(a, b):
        return jax.lax.dot(a, b, preferred_element_type=jnp.float32)

    def dot_split(a, b):
        hi, lo = _bf16_split(a)
        return dot(hi, b) + dot(lo, b)

    xb, xr = _bf16_split(x)
    tau_i = jnp.max(x, axis=0, keepdims=True) - 1.0
    tau_t = (dot(xb, M) + dot(xr, M) - 1.0) / jnp.float32(_LST)

    for it in range(max(_ITERS_INST, _ITERS_TIME)):
        if it < _ITERS_INST:
            mask = (x > tau_i).astype(dt)
            S = jnp.sum(x * mask, axis=0, keepdims=True)
            C = jnp.sum(mask, axis=0, keepdims=True)
            tau_i = jnp.where(C > 0.0, (S - 1.0) / jnp.maximum(C, 1.0), tau_i)
        if it < _ITERS_TIME:
            tau_b = dot_split(tau_t, Mt)
            mask = (x > tau_b).astype(dt)
            S = dot(xb * mask, M) + dot(xr * mask, M)
            C = dot(mask, M)
            tau_t = jnp.where(C > 0.0, (S - 1.0) / jnp.maximum(C, 1.0), tau_t)

    tau_tb = dot_split(tau_t, Mt)

    o_ref[0] = jnp.maximum(x - tau_i, 0.0) * jnp.maximum(x - tau_tb, 0.0)


def kernel(midis_out):
    batch, two, n_insts, time = midis_out.shape
    assert time % _LST == 0

    bc = batch * two
    x3 = midis_out.reshape(bc, n_insts, time)

    T_BLK = 4096
    out = pl.pallas_call(
        _fused_kernel,
        grid=(bc, time // T_BLK),
        in_specs=[pl.BlockSpec((1, n_insts, T_BLK), lambda i, j: (i, 0, j))],
        out_specs=pl.BlockSpec((1, n_insts, T_BLK), lambda i, j: (i, 0, j)),
        out_shape=jax.ShapeDtypeStruct(x3.shape, x3.dtype),
    )(x3)

    return out.reshape(batch, two, n_insts, time)
